# baseline (speedup 1.0000x reference)
import jax
import jax.numpy as jnp
from jax.experimental import pallas as pl
from jax.experimental.pallas import tpu as pltpu

_DN = (((1,), (1,)), ((), ()))


def _sage_block_kernel(adj0_ref, adj1_ref, feat_ref, w_ref, b_ref, out_ref):
    i = pl.program_id(0)
    d = out_ref.shape[1]
    hm = adj0_ref.shape[0]
    for s, adj_ref in enumerate((adj0_ref, adj1_ref)):
        nb = jnp.dot(adj_ref[...], feat_ref[...], preferred_element_type=jnp.float32)
        self_f = feat_ref[pl.ds((2 * i + s) * hm, hm), :]
        out = (
            jax.lax.dot_general(
                self_f, w_ref[:, 0:d], _DN, preferred_element_type=jnp.float32
            )
            + jax.lax.dot_general(
                nb, w_ref[:, d : 2 * d], _DN, preferred_element_type=jnp.float32
            )
            + b_ref[...]
        )
        norm = jnp.sqrt(jnp.sum(out * out, axis=1, keepdims=True))
        out_ref[pl.ds(s * hm, hm), :] = out / jnp.maximum(norm, 1e-12)


def kernel(features, adj, W, b):
    n, d = features.shape
    bm = 400
    hm = bm // 2
    b2 = b.reshape(1, d)
    return pl.pallas_call(
        _sage_block_kernel,
        grid=(n // bm,),
        in_specs=[
            pl.BlockSpec((hm, n), lambda i: (2 * i, 0)),
            pl.BlockSpec((hm, n), lambda i: (2 * i + 1, 0)),
            pl.BlockSpec((n, d), lambda i: (0, 0)),
            pl.BlockSpec((d, 2 * d), lambda i: (0, 0)),
            pl.BlockSpec((1, d), lambda i: (0, 0)),
        ],
        out_specs=pl.BlockSpec((bm, d), lambda i: (i, 0)),
        out_shape=jax.ShapeDtypeStruct((n, d), jnp.float32),
        compiler_params=pltpu.CompilerParams(
            dimension_semantics=("arbitrary",),
            vmem_limit_bytes=100 * 1024 * 1024,
        ),
    )(adj, adj, features, W, b2)


# speedup vs baseline: 1.1211x; 1.1211x over previous
import jax
import jax.numpy as jnp
from jax.experimental import pallas as pl
from jax.experimental.pallas import tpu as pltpu

_DN = (((1,), (1,)), ((), ()))


def _sage_block_kernel(adj_ref, feat_ref, w_ref, b_ref, out_ref):
    i = pl.program_id(0)
    bm, d = out_ref.shape
    nb = jnp.dot(adj_ref[...], feat_ref[...], preferred_element_type=jnp.float32)
    self_f = feat_ref[pl.ds(i * bm, bm), :]
    out = (
        jax.lax.dot_general(
            self_f, w_ref[:, 0:d], _DN, preferred_element_type=jnp.float32
        )
        + jax.lax.dot_general(
            nb, w_ref[:, d : 2 * d], _DN, preferred_element_type=jnp.float32
        )
        + b_ref[...]
    )
    norm = jnp.sqrt(jnp.sum(out * out, axis=1, keepdims=True))
    out_ref[...] = out / jnp.maximum(norm, 1e-12)


def kernel(features, adj, W, b):
    n, d = features.shape
    bm = 400
    b2 = b.reshape(1, d)
    return pl.pallas_call(
        _sage_block_kernel,
        grid=(n // bm,),
        in_specs=[
            pl.BlockSpec((bm, n), lambda i: (i, 0)),
            pl.BlockSpec((n, d), lambda i: (0, 0)),
            pl.BlockSpec((d, 2 * d), lambda i: (0, 0)),
            pl.BlockSpec((1, d), lambda i: (0, 0)),
        ],
        out_specs=pl.BlockSpec((bm, d), lambda i: (i, 0)),
        out_shape=jax.ShapeDtypeStruct((n, d), jnp.float32),
        compiler_params=pltpu.CompilerParams(
            dimension_semantics=("arbitrary",),
            vmem_limit_bytes=100 * 1024 * 1024,
        ),
    )(adj, features, W, b2)
